# baseline (speedup 1.0000x reference)
import functools

import jax
import jax.numpy as jnp
from jax import lax
from jax.experimental import pallas as pl
from jax.experimental.pallas import tpu as pltpu
from jax.experimental.pallas import tpu_sc as plsc

BATCH = 16384
EMBED = 32
NUM_CORES = 2
NUM_SUBCORES = 16
NW = NUM_CORES * NUM_SUBCORES
BPW = BATCH // NW
LANES = 16
CHUNK = 256
GROUP = 128 // EMBED
TCOLS = 512


def _tc_regroup_body(in_ref, out_ref):
    x = in_ref[...]
    kpc = 128 // GROUP
    for c in range(TCOLS // 128):
        xc = x[:, c * 128:(c + 1) * 128]
        parts = []
        for p in range(GROUP):
            lanes = jnp.broadcast_to(
                (GROUP * jnp.arange(kpc) + p)[None, :], (EMBED, kpc))
            parts.append(jnp.take_along_axis(xc, lanes, axis=1))
        r = jnp.concatenate(parts, axis=0)
        out_ref[pl.ds(c * kpc, kpc), :] = r.T


def _tc_regroup(table_t):
    v = table_t.shape[1]
    return pl.pallas_call(
        _tc_regroup_body,
        grid=(pl.cdiv(v, TCOLS),),
        in_specs=[pl.BlockSpec((EMBED, TCOLS), lambda i: (0, i))],
        out_specs=pl.BlockSpec((TCOLS // GROUP, 128), lambda i: (i, 0)),
        out_shape=jax.ShapeDtypeStruct((v // GROUP, 128), jnp.float32),
    )(table_t)


def _sc_forward(user_id, ad_id, u_r, a_r, wb):
    mesh = plsc.VectorSubcoreMesh(core_axis_name="c", subcore_axis_name="s")

    @functools.partial(
        pl.kernel,
        out_type=jax.ShapeDtypeStruct((BATCH,), jnp.float32),
        mesh=mesh,
        scratch_types=[
            pltpu.VMEM((BPW,), jnp.int32),
            pltpu.VMEM((BPW,), jnp.int32),
            pltpu.VMEM((CHUNK,), jnp.int32),
            pltpu.VMEM((CHUNK,), jnp.int32),
            pltpu.VMEM((CHUNK, 128), jnp.float32),
            pltpu.VMEM((CHUNK, 128), jnp.float32),
            pltpu.VMEM((2, LANES), jnp.float32),
            pltpu.VMEM((BPW,), jnp.float32),
            pltpu.SemaphoreType.DMA,
            pltpu.SemaphoreType.DMA,
        ],
        compiler_params=pltpu.CompilerParams(use_tc_tiling_on_sc=True,
                                             needs_layout_passes=False),
        cost_estimate=pl.CostEstimate(
            flops=2 * BATCH * EMBED,
            transcendentals=BATCH,
            bytes_accessed=2 * BATCH * 128 * 4,
        ),
    )
    def k(uid_hbm, aid_hbm, ut_hbm, at_hbm, wb_hbm, out_hbm,
          uid_v, aid_v, ug_v, ag_v, urows_v, arows_v, wb_v, dots_v,
          sem_u, sem_a):
        wid = lax.axis_index("s") * NUM_CORES + lax.axis_index("c")
        base = wid * BPW
        pltpu.sync_copy(uid_hbm.at[pl.ds(base, BPW)], uid_v)
        pltpu.sync_copy(aid_hbm.at[pl.ds(base, BPW)], aid_v)
        pltpu.sync_copy(wb_hbm, wb_v)
        w = wb_v[0, :]
        b = wb_v[1, :]

        @pl.loop(0, BPW, step=CHUNK)
        def _(c0):
            @pl.loop(0, CHUNK, step=LANES)
            def _(jb):
                uvec = uid_v[pl.ds(c0 + jb, LANES)]
                avec = aid_v[pl.ds(c0 + jb, LANES)]
                ug_v[pl.ds(jb, LANES)] = jax.lax.shift_right_logical(uvec, 2)
                ag_v[pl.ds(jb, LANES)] = jax.lax.shift_right_logical(avec, 2)

            cu = pltpu.async_copy(ut_hbm.at[ug_v], urows_v, sem_u)
            ca = pltpu.async_copy(at_hbm.at[ag_v], arows_v, sem_a)
            cu.wait()
            ca.wait()

            @pl.loop(0, CHUNK, step=LANES)
            def _(jb):
                uvec = uid_v[pl.ds(c0 + jb, LANES)]
                avec = aid_v[pl.ds(c0 + jb, LANES)]
                uoff = (uvec & (GROUP - 1)) * EMBED
                aoff = (avec & (GROUP - 1)) * EMBED
                rows = jax.lax.iota(jnp.int32, LANES) + jb
                acc = plsc.load_gather(urows_v, [rows, uoff]) * \
                    plsc.load_gather(arows_v, [rows, aoff])
                for e in range(1, EMBED):
                    acc += plsc.load_gather(urows_v, [rows, uoff + e]) * \
                        plsc.load_gather(arows_v, [rows, aoff + e])
                z = acc * w + b
                dots_v[pl.ds(c0 + jb, LANES)] = 1.0 / (1.0 + jnp.exp(-z))

        pltpu.sync_copy(dots_v, out_hbm.at[pl.ds(base, BPW)])

    return k(user_id, ad_id, u_r, a_r, wb)


def kernel(user_id, ad_id, user_table, ad_table, fc_w, fc_b):
    u_r = _tc_regroup(user_table.T)
    a_r = _tc_regroup(ad_table.T)
    w = fc_w.reshape(())
    b = fc_b.reshape(())
    wb = jnp.stack([jnp.broadcast_to(w, (LANES,)),
                    jnp.broadcast_to(b, (LANES,))])
    out = _sc_forward(user_id, ad_id, u_r, a_r, wb)
    return out.reshape(BATCH, 1)
